# baseline (speedup 1.0000x reference)
import functools

import jax
import jax.numpy as jnp
from jax.experimental import pallas as pl
from jax.experimental.pallas import tpu as pltpu

_MARGIN = 1.0
_N = 65536
_D = 32
_K = 5000
_KPAD = 5120
_RB = 512
_CB = 2048
_NRB = _KPAD // _RB
_NCB = _N // _CB


def _tc_body(scal_ref, rows_ref, emb_ref, lab_ref, out_ref):
    i = pl.program_id(0)
    j = pl.program_id(1)
    a = rows_ref[...]
    b = emb_ref[...]
    lab = lab_ref[0, 0, :]
    count_f = scal_ref[0]
    n_pos = scal_ref[1]
    pos_sum = scal_ref[2]

    a2 = jnp.sum(a * a, axis=1)[:, None]
    b2 = jnp.sum(b * b, axis=1)[None, :]
    ab = jax.lax.dot_general(a, b, (((1,), (1,)), ((), ())),
                             preferred_element_type=jnp.float32)
    d2 = jnp.maximum(a2 + b2 - 2.0 * ab, 1e-12)
    m = jnp.maximum(_MARGIN - jnp.sqrt(d2), 0.0)

    rowid = (jnp.float32(i * _RB)
             + jax.lax.broadcasted_iota(jnp.int32, (_RB, 1), 0)
             .astype(jnp.float32))
    rv = jnp.where((rowid < count_f) & (rowid < jnp.float32(_K)), 1.0, 0.0)
    pf = jnp.where(lab != 0, 1.0, 0.0)[None, :]
    partial = jnp.sum(m * m * rv * pf, keepdims=True)

    @pl.when((i == 0) & (j == 0))
    def _init():
        out_ref[...] = jnp.zeros((1, 1), jnp.float32)

    out_ref[...] += partial

    @pl.when((i == _NRB - 1) & (j == _NCB - 1))
    def _finish():
        total = out_ref[...]
        n_hard = jnp.minimum(count_f, jnp.float32(_K))
        neg = jnp.where(count_f > 0.0, total / (n_hard * n_pos),
                        jnp.zeros((1, 1), jnp.float32))
        out_ref[...] = pos_sum / n_pos + neg


_tc_call = pl.pallas_call(
    _tc_body,
    grid=(_NRB, _NCB),
    in_specs=[
        pl.BlockSpec(memory_space=pltpu.SMEM),
        pl.BlockSpec((_RB, _D), lambda i, j: (i, 0)),
        pl.BlockSpec((_CB, _D), lambda i, j: (j, 0)),
        pl.BlockSpec((1, 1, _CB), lambda i, j: (j, 0, 0)),
    ],
    out_specs=pl.BlockSpec((1, 1), lambda i, j: (0, 0)),
    out_shape=jax.ShapeDtypeStruct((1, 1), jnp.float32),
)


def kernel(data, embeddings, labels):
    pos_mask = labels != 0
    n_pos = jnp.sum(pos_mask.astype(jnp.float32))
    pos_sum = jnp.sum(data * data * pos_mask.astype(jnp.float32))
    hard = (labels == 0) & (data < _MARGIN)
    count = jnp.sum(hard.astype(jnp.int32))
    (idx,) = jnp.nonzero(hard, size=_K, fill_value=0)
    idx = jnp.pad(idx, (0, _KPAD - _K)).astype(jnp.int32)
    rows = embeddings[idx]

    scal = jnp.zeros((16,), jnp.float32)
    scal = scal.at[0].set(count.astype(jnp.float32))
    scal = scal.at[1].set(n_pos)
    scal = scal.at[2].set(pos_sum)

    labels3 = labels.reshape(_NCB, 1, _CB)
    out = _tc_call(scal, rows, embeddings, labels3)
    return out[0, 0]
